# Initial kernel scaffold; baseline (speedup 1.0000x reference)
import jax
import jax.numpy as jnp
from jax.experimental import pallas as pl


def kernel(x, edge_index, batch, W1, b1, W2, b2, gamma, beta):
    raise NotImplementedError("write your pallas kernel here")


# speedup vs baseline: 3.8293x; 3.8293x over previous
import functools

import jax
import jax.numpy as jnp
from jax import lax
from jax.experimental import pallas as pl
from jax.experimental.pallas import tpu as pltpu
from jax.experimental.pallas import tpu_sc as plsc

N = 10000
D = 128
E = 320000
NPAD = 10112
ROWS_PER_TILE = NPAD // 16
K = 128
NW = 32
CHUNKS = -(-E // (NW * K))
EPW = CHUNKS * K
EPAD = NW * EPW


def _agg_body(h_hbm, src_hbm, dst_hbm, zeros_hbm, out_hbm,
              src_v, dst_v, rows_v, sem, acc):
    c = lax.axis_index("c")
    s = lax.axis_index("s")
    wid = s * 2 + c
    r0 = s * ROWS_PER_TILE
    pltpu.sync_copy(zeros_hbm.at[pl.ds(r0, ROWS_PER_TILE)],
                    acc.at[pl.ds(r0, ROWS_PER_TILE)])
    plsc.subcore_barrier()

    base = wid * EPW

    def body(i, carry):
        off = base + i * K
        pltpu.sync_copy(src_hbm.at[pl.ds(off, K)], src_v)
        pltpu.sync_copy(dst_hbm.at[pl.ds(off, K)], dst_v)
        pltpu.async_copy(h_hbm.at[src_v], rows_v, sem).wait()
        pltpu.sync_copy(rows_v, acc.at[dst_v], add=True)
        return carry

    lax.fori_loop(0, CHUNKS, body, 0)
    plsc.subcore_barrier()
    pltpu.sync_copy(acc.at[pl.ds(r0, ROWS_PER_TILE)],
                    out_hbm.at[c, pl.ds(r0, ROWS_PER_TILE)])


@functools.cache
def _make_agg():
    return pl.kernel(
        _agg_body,
        out_type=jax.ShapeDtypeStruct((2, NPAD, D), jnp.float32),
        mesh=plsc.VectorSubcoreMesh(core_axis_name="c", subcore_axis_name="s"),
        scratch_types=[
            pltpu.VMEM((K,), jnp.int32),
            pltpu.VMEM((K,), jnp.int32),
            pltpu.VMEM((K, D), jnp.float32),
            pltpu.SemaphoreType.DMA,
            pltpu.VMEM_SHARED((NPAD, D), jnp.float32),
        ],
    )


def _bn(z, g, b):
    mu = jnp.mean(z, axis=0, keepdims=True)
    var = jnp.mean((z - mu) ** 2, axis=0, keepdims=True)
    return (z - mu) * lax.rsqrt(var + 1e-5) * g + b


def _mlp(z, w1, b1, w2, b2):
    z = jnp.maximum(
        jnp.dot(z, w1, preferred_element_type=jnp.float32) + b1, 0.0)
    return jnp.dot(z, w2, preferred_element_type=jnp.float32) + b2


def _combine_body(h_ref, p_ref, z_ref):
    z_ref[...] = h_ref[...] + p_ref[0, :N, :] + p_ref[1, :N, :]


_combine = pl.pallas_call(
    _combine_body,
    out_shape=jax.ShapeDtypeStruct((N, D), jnp.float32),
)


def _layer_body(z_ref, w1_ref, b1_ref, w2_ref, b2_ref, g_ref, be_ref,
                out_ref, *, act):
    z = _mlp(z_ref[...], w1_ref[...], b1_ref[...], w2_ref[...], b2_ref[...])
    z = jnp.maximum(z, 0.0) if act == "relu" else jnp.tanh(z)
    out_ref[...] = _bn(z, g_ref[...], be_ref[...])


_layer = pl.pallas_call(
    functools.partial(_layer_body, act="relu"),
    out_shape=jax.ShapeDtypeStruct((N, D), jnp.float32),
)

_head = pl.pallas_call(
    functools.partial(_layer_body, act="tanh"),
    out_shape=jax.ShapeDtypeStruct((N, D), jnp.float32),
)


def kernel(x, edge_index, batch, W1, b1, W2, b2, gamma, beta):
    src = edge_index[0].astype(jnp.int32)
    dst = edge_index[1].astype(jnp.int32)
    pad = EPAD - E
    src_p = jnp.concatenate([src, jnp.zeros((pad,), jnp.int32)])
    dst_p = jnp.concatenate([dst, jnp.full((pad,), N, jnp.int32)])
    zeros = jnp.zeros((NPAD, D), jnp.float32)

    b1r = b1.reshape(-1, 1, D)
    b2r = b2.reshape(-1, 1, D)
    gr = gamma.reshape(-1, 1, D)
    ber = beta.reshape(-1, 1, D)

    agg = _make_agg()
    h = x
    for i in range(3):
        p = agg(h, src_p, dst_p, zeros)
        z = _combine(h, p)
        h = _layer(z, W1[i], b1r[i], W2[i], b2r[i], gr[i], ber[i])
    p = agg(h, src_p, dst_p, zeros)
    z = _combine(h, p)
    return tuple(_head(z, W1[j], b1r[j], W2[j], b2r[j], gr[j], ber[j])
                 for j in range(3, 7))
